# baseline (speedup 1.0000x reference)
import functools

import jax
import jax.numpy as jnp
from jax import lax
from jax.experimental import pallas as pl
from jax.experimental.pallas import tpu as pltpu
from jax.experimental.pallas import tpu_sc as plsc

V = 100000
E = 128
H = 256
B = 1024
T = 50
G4 = 4 * H

NC, NS = 2, 16
NW = NC * NS
BT = B * T
B_PER_W = BT // NW
CHUNK = 400
N_CHUNK = B_PER_W // CHUNK


def _sc_gather(emb, idx_flat):
    mesh = plsc.VectorSubcoreMesh(core_axis_name="c", subcore_axis_name="s")

    @functools.partial(
        pl.kernel,
        mesh=mesh,
        out_type=jax.ShapeDtypeStruct((BT, E), jnp.float32),
        scratch_types=[
            pltpu.VMEM((CHUNK,), jnp.int32),
            pltpu.VMEM((CHUNK, E), jnp.float32),
            pltpu.SemaphoreType.DMA,
        ],
    )
    def gather_k(table_hbm, idx_hbm, out_hbm, idx_v, rows_v, sem):
        wid = lax.axis_index("s") * NC + lax.axis_index("c")
        base = wid * B_PER_W

        def body(j, carry):
            off = base + j * CHUNK
            pltpu.sync_copy(idx_hbm.at[pl.ds(off, CHUNK)], idx_v)
            pltpu.async_copy(table_hbm.at[idx_v], rows_v, sem).wait()
            pltpu.sync_copy(rows_v, out_hbm.at[pl.ds(off, CHUNK)])
            return carry

        lax.fori_loop(0, N_CHUNK, body, 0)

    return gather_k(emb, idx_flat)


def _sigmoid(z):
    return 0.5 * jnp.tanh(0.5 * z) + 0.5


def _cell(xh, c_old, w_r, b_r):
    gates = lax.dot_general(xh, w_r[...], (((1,), (1,)), ((), ())),
                            preferred_element_type=jnp.float32) + b_r[...]
    i = _sigmoid(gates[:, 0:H])
    f = _sigmoid(gates[:, H:2 * H])
    g = jnp.tanh(gates[:, 2 * H:3 * H])
    o = _sigmoid(gates[:, 3 * H:4 * H])
    c = f * c_old + i * g
    h = o * jnp.tanh(c)
    return h, c


def _fused_lstm(e, w0, b0, w1, b1, w2, b2, ln_g, ln_b, interpret=False):
    bf = jnp.bfloat16

    EH = E + H
    H2 = 2 * H

    def body(x_ref, w0_r, b0_r, w1_r, b1_r, w2_r, b2_r, lng_r, lnb_r,
             h0T, c0T, h1T, c1T, h2T, c2T, nrm,
             nrm16,
             xh0s, h01s, h12s, c0s, c1s, c2s):
        i = pl.program_id(0)

        @pl.when(i == 0)
        def _init():
            xh0s[0] = jnp.zeros((B, EH), bf)
            h01s[0] = jnp.zeros((B, H2), bf)
            h12s[0] = jnp.zeros((B, H2), bf)
            for r in (c0s, c1s, c2s):
                r[...] = jnp.zeros((B, H), jnp.float32)

        def substep(s, x_sub, rp, wq):
            xh0s[rp, :, 0:E] = x_sub.astype(bf)
            ha, ca = _cell(xh0s[rp], c0s[...], w0_r, b0_r)
            hb, cb = _cell(h01s[rp], c1s[...], w1_r, b1_r)
            hc, cc = _cell(h12s[rp], c2s[...], w2_r, b2_r)

            h0b = ha.astype(bf)
            h1b = hb.astype(bf)
            xh0s[wq, :, E:EH] = h0b
            h01s[wq, :, 0:H] = h0b
            h01s[wq, :, H:H2] = h1b
            h12s[wq, :, 0:H] = h1b
            h12s[wq, :, H:H2] = hc.astype(bf)
            c0s[...] = ca
            c1s[...] = cb
            c2s[...] = cc

            @pl.when(s == T - 1)
            def _cap0():
                h0T[...] = ha
                c0T[...] = ca

            @pl.when(s == T)
            def _cap1():
                h1T[...] = hb
                c1T[...] = cb

            @pl.when(s == T + 1)
            def _fin():
                h2T[...] = hc
                c2T[...] = cc
                mu = jnp.mean(hc, axis=-1, keepdims=True)
                var = jnp.mean((hc - mu) ** 2, axis=-1, keepdims=True)
                n = ((hc - mu) * lax.rsqrt(var + 1e-5)
                     * lng_r[...] + lnb_r[...])
                nrm[...] = n
                nrm16[...] = n.astype(bf)

        s0 = 2 * i
        substep(s0, x_ref[0], 0, 1)

        @pl.when(i == 0)
        def _rz0():
            h01s[1, :, H:H2] = jnp.zeros((B, H), bf)
            h12s[1] = jnp.zeros((B, H2), bf)
            c1s[...] = jnp.zeros((B, H), jnp.float32)
            c2s[...] = jnp.zeros((B, H), jnp.float32)

        substep(s0 + 1, x_ref[1], 1, 0)

        @pl.when(i == 0)
        def _rz1():
            h12s[0, :, H:H2] = jnp.zeros((B, H), bf)
            c2s[...] = jnp.zeros((B, H), jnp.float32)

    const = lambda s: (0, 0)
    bh = pl.BlockSpec((B, H), const)
    return pl.pallas_call(
        body,
        grid=((T + 2) // 2,),
        in_specs=[
            pl.BlockSpec((2, B, E),
                         lambda i: (jnp.minimum(i, T // 2 - 1), 0, 0)),
            pl.BlockSpec((G4, E + H), const),
            pl.BlockSpec((1, G4), const),
            pl.BlockSpec((G4, 2 * H), const),
            pl.BlockSpec((1, G4), const),
            pl.BlockSpec((G4, 2 * H), const),
            pl.BlockSpec((1, G4), const),
            pl.BlockSpec((1, H), const),
            pl.BlockSpec((1, H), const),
        ],
        out_specs=[bh] * 8,
        out_shape=[jax.ShapeDtypeStruct((B, H), jnp.float32)] * 7
        + [jax.ShapeDtypeStruct((B, H), jnp.bfloat16)],
        scratch_shapes=[
            pltpu.VMEM((2, B, EH), bf),
            pltpu.VMEM((2, B, H2), bf),
            pltpu.VMEM((2, B, H2), bf),
            pltpu.VMEM((B, H), jnp.float32),
            pltpu.VMEM((B, H), jnp.float32),
            pltpu.VMEM((B, H), jnp.float32),
        ],
        interpret=interpret,
    )(e, w0, b0, w1, b1, w2, b2, ln_g, ln_b)


VB = 2048
NVB = -(-V // VB)
VPAD = NVB * VB


def _head(normed, lin_W, lin_b_pad, interpret=False):
    def body(n_ref, w_ref, b_ref, out_ref):
        acc = lax.dot_general(w_ref[...].astype(jnp.bfloat16),
                              n_ref[...],
                              (((1,), (1,)), ((), ())),
                              preferred_element_type=jnp.float32)
        out_ref[...] = acc + jnp.transpose(b_ref[0])

    return pl.pallas_call(
        body,
        grid=(NVB,),
        in_specs=[
            pl.BlockSpec((B, H), lambda i: (0, 0)),
            pl.BlockSpec((VB, H), lambda i: (i, 0)),
            pl.BlockSpec((1, 1, VB), lambda i: (i, 0, 0)),
        ],
        out_specs=pl.BlockSpec((VB, B), lambda i: (i, 0)),
        out_shape=jax.ShapeDtypeStruct((V, B), jnp.float32),
        interpret=interpret,
    )(normed, lin_W, lin_b_pad)


def kernel(x, emb, W_ih0, W_hh0, b_ih0, b_hh0, W_ih1, W_hh1, b_ih1, b_hh1,
           W_ih2, W_hh2, b_ih2, b_hh2, ln_g, ln_b, lin_W, lin_b):
    idx_flat = x.T.reshape(BT)
    e = _sc_gather(emb, idx_flat).reshape(T, B, E)

    bf = jnp.bfloat16
    w0 = jnp.concatenate([W_ih0, W_hh0], axis=1).astype(bf)
    w1 = jnp.concatenate([W_ih1, W_hh1], axis=1).astype(bf)
    w2 = jnp.concatenate([W_ih2, W_hh2], axis=1).astype(bf)
    h0, c0, h1, c1, h2, c2, normed, normed16 = _fused_lstm(
        e,
        w0, (b_ih0 + b_hh0).reshape(1, G4),
        w1, (b_ih1 + b_hh1).reshape(1, G4),
        w2, (b_ih2 + b_hh2).reshape(1, G4),
        ln_g.reshape(1, H), ln_b.reshape(1, H))

    lin_b_pad = jnp.zeros((VPAD,), jnp.float32).at[:V].set(lin_b)
    logits = _head(normed16, lin_W, lin_b_pad.reshape(NVB, 1, VB)).T

    hidden = (jnp.stack([h0, h1, h2]), jnp.stack([c0, c1, c2]))
    return logits, hidden


# speedup vs baseline: 1.0297x; 1.0297x over previous
import functools

import jax
import jax.numpy as jnp
from jax import lax
from jax.experimental import pallas as pl
from jax.experimental.pallas import tpu as pltpu
from jax.experimental.pallas import tpu_sc as plsc

V = 100000
E = 128
H = 256
B = 1024
T = 50
L = 3
G4 = 4 * H

NC, NS = 2, 16
NW = NC * NS
BT = B * T
B_PER_W = BT // NW
CHUNK = 400
N_CHUNK = B_PER_W // CHUNK


def _sc_gather(emb, idx_flat):
    mesh = plsc.VectorSubcoreMesh(core_axis_name="c", subcore_axis_name="s")

    @functools.partial(
        pl.kernel,
        mesh=mesh,
        out_type=jax.ShapeDtypeStruct((BT, E), jnp.float32),
        scratch_types=[
            pltpu.VMEM((CHUNK,), jnp.int32),
            pltpu.VMEM((CHUNK, E), jnp.float32),
            pltpu.SemaphoreType.DMA,
        ],
    )
    def gather_k(table_hbm, idx_hbm, out_hbm, idx_v, rows_v, sem):
        wid = lax.axis_index("s") * NC + lax.axis_index("c")
        base = wid * B_PER_W

        def body(j, carry):
            off = base + j * CHUNK
            pltpu.sync_copy(idx_hbm.at[pl.ds(off, CHUNK)], idx_v)
            pltpu.async_copy(table_hbm.at[idx_v], rows_v, sem).wait()
            pltpu.sync_copy(rows_v, out_hbm.at[pl.ds(off, CHUNK)])
            return carry

        lax.fori_loop(0, N_CHUNK, body, 0)

    return gather_k(emb, idx_flat)


def _sigmoid(z):
    return 0.5 * jnp.tanh(0.5 * z) + 0.5


def _cell(xh, c_old, w_r, b_r):
    gates = lax.dot_general(xh, w_r[...], (((1,), (1,)), ((), ())),
                            preferred_element_type=jnp.float32) + b_r[...]
    i = _sigmoid(gates[:, 0:H])
    f = _sigmoid(gates[:, H:2 * H])
    g = jnp.tanh(gates[:, 2 * H:3 * H])
    o = _sigmoid(gates[:, 3 * H:4 * H])
    c = f * c_old + i * g
    h = o * jnp.tanh(c)
    return h, c


def _fused_lstm(e, w0, b0, w1, b1, w2, b2, ln_g, ln_b, interpret=False):
    bf = jnp.bfloat16

    def body(x_ref, w0_r, b0_r, w1_r, b1_r, w2_r, b2_r, lng_r, lnb_r,
             hT, cT, nrm16,
             h0s, h1s, h2s, c0s, c1s, c2s):
        s = pl.program_id(0)

        @pl.when(s == 0)
        def _init():
            for r in (h0s, h1s, h2s):
                r[...] = jnp.zeros((B, H), bf)
            for r in (c0s, c1s, c2s):
                r[...] = jnp.zeros((B, H), jnp.float32)

        h0o = h0s[...]
        h1o = h1s[...]

        ha, ca = _cell(jnp.concatenate([x_ref[0].astype(bf), h0o], axis=1),
                       c0s[...], w0_r, b0_r)
        hb, cb = _cell(jnp.concatenate([h0o, h1o], axis=1),
                       c1s[...], w1_r, b1_r)
        hc, cc = _cell(jnp.concatenate([h1o, h2s[...]], axis=1),
                       c2s[...], w2_r, b2_r)

        h0s[...] = ha.astype(bf)
        c0s[...] = ca
        h1s[...] = hb.astype(bf)
        c1s[...] = cb
        h2s[...] = hc.astype(bf)
        c2s[...] = cc

        @pl.when(s == 0)
        def _rz0():
            for r in (h1s, h2s):
                r[...] = jnp.zeros((B, H), bf)
            for r in (c1s, c2s):
                r[...] = jnp.zeros((B, H), jnp.float32)

        @pl.when(s == 1)
        def _rz1():
            h2s[...] = jnp.zeros((B, H), bf)
            c2s[...] = jnp.zeros((B, H), jnp.float32)

        @pl.when(s == T - 1)
        def _cap0():
            hT[0] = ha
            cT[0] = ca

        @pl.when(s == T)
        def _cap1():
            hT[1] = hb
            cT[1] = cb

        @pl.when(s == T + 1)
        def _fin():
            hT[2] = hc
            cT[2] = cc
            mu = jnp.mean(hc, axis=-1, keepdims=True)
            var = jnp.mean((hc - mu) ** 2, axis=-1, keepdims=True)
            n = ((hc - mu) * lax.rsqrt(var + 1e-5)
                 * lng_r[...] + lnb_r[...])
            nrm16[...] = n.astype(bf)

    const = lambda s: (0, 0)
    return pl.pallas_call(
        body,
        grid=(T + 2,),
        in_specs=[
            pl.BlockSpec((1, B, E), lambda s: (jnp.minimum(s, T - 1), 0, 0)),
            pl.BlockSpec((G4, E + H), const),
            pl.BlockSpec((1, G4), const),
            pl.BlockSpec((G4, 2 * H), const),
            pl.BlockSpec((1, G4), const),
            pl.BlockSpec((G4, 2 * H), const),
            pl.BlockSpec((1, G4), const),
            pl.BlockSpec((1, H), const),
            pl.BlockSpec((1, H), const),
        ],
        out_specs=[
            pl.BlockSpec((L, B, H), lambda s: (0, 0, 0)),
            pl.BlockSpec((L, B, H), lambda s: (0, 0, 0)),
            pl.BlockSpec((B, H), const),
        ],
        out_shape=[
            jax.ShapeDtypeStruct((L, B, H), jnp.float32),
            jax.ShapeDtypeStruct((L, B, H), jnp.float32),
            jax.ShapeDtypeStruct((B, H), jnp.bfloat16),
        ],
        scratch_shapes=[pltpu.VMEM((B, H), bf)] * 3
        + [pltpu.VMEM((B, H), jnp.float32)] * 3,
        interpret=interpret,
    )(e, w0, b0, w1, b1, w2, b2, ln_g, ln_b)


VB = 2048
NVB = -(-V // VB)
VPAD = NVB * VB


def _head(normed, lin_W, lin_b_pad, interpret=False):
    def body(n_ref, w_ref, b_ref, out_ref):
        acc = lax.dot_general(w_ref[...].astype(jnp.bfloat16),
                              n_ref[...],
                              (((1,), (1,)), ((), ())),
                              preferred_element_type=jnp.float32)
        out_ref[...] = acc + jnp.transpose(b_ref[0])

    return pl.pallas_call(
        body,
        grid=(NVB,),
        in_specs=[
            pl.BlockSpec((B, H), lambda i: (0, 0)),
            pl.BlockSpec((VB, H), lambda i: (i, 0)),
            pl.BlockSpec((1, 1, VB), lambda i: (i, 0, 0)),
        ],
        out_specs=pl.BlockSpec((VB, B), lambda i: (i, 0)),
        out_shape=jax.ShapeDtypeStruct((V, B), jnp.float32),
        interpret=interpret,
    )(normed, lin_W, lin_b_pad)


def kernel(x, emb, W_ih0, W_hh0, b_ih0, b_hh0, W_ih1, W_hh1, b_ih1, b_hh1,
           W_ih2, W_hh2, b_ih2, b_hh2, ln_g, ln_b, lin_W, lin_b):
    idx_flat = x.T.reshape(BT)
    e = _sc_gather(emb, idx_flat).reshape(T, B, E)

    bf = jnp.bfloat16
    w0 = jnp.concatenate([W_ih0, W_hh0], axis=1).astype(bf)
    w1 = jnp.concatenate([W_ih1, W_hh1], axis=1).astype(bf)
    w2 = jnp.concatenate([W_ih2, W_hh2], axis=1).astype(bf)
    hT, cT, normed16 = _fused_lstm(
        e,
        w0, (b_ih0 + b_hh0).reshape(1, G4),
        w1, (b_ih1 + b_hh1).reshape(1, G4),
        w2, (b_ih2 + b_hh2).reshape(1, G4),
        ln_g.reshape(1, H), ln_b.reshape(1, H))

    lin_b_pad = jnp.zeros((VPAD,), jnp.float32).at[:V].set(lin_b)
    logits = _head(normed16, lin_W, lin_b_pad.reshape(NVB, 1, VB)).T

    return logits, (hT, cT)


# speedup vs baseline: 1.0382x; 1.0082x over previous
import functools

import jax
import jax.numpy as jnp
from jax import lax
from jax.experimental import pallas as pl
from jax.experimental.pallas import tpu as pltpu
from jax.experimental.pallas import tpu_sc as plsc

V = 100000
E = 128
H = 256
B = 1024
T = 50
L = 3
G4 = 4 * H

NC, NS = 2, 16
NW = NC * NS
BT = B * T
B_PER_W = BT // NW
CHUNK = 400
N_CHUNK = B_PER_W // CHUNK


def _sc_gather(emb, idx_flat):
    mesh = plsc.VectorSubcoreMesh(core_axis_name="c", subcore_axis_name="s")

    @functools.partial(
        pl.kernel,
        mesh=mesh,
        out_type=jax.ShapeDtypeStruct((BT, E), jnp.float32),
        scratch_types=[
            pltpu.VMEM((CHUNK,), jnp.int32),
            pltpu.VMEM((CHUNK, E), jnp.float32),
            pltpu.SemaphoreType.DMA,
        ],
    )
    def gather_k(table_hbm, idx_hbm, out_hbm, idx_v, rows_v, sem):
        wid = lax.axis_index("s") * NC + lax.axis_index("c")
        base = wid * B_PER_W

        def body(j, carry):
            off = base + j * CHUNK
            pltpu.sync_copy(idx_hbm.at[pl.ds(off, CHUNK)], idx_v)
            pltpu.async_copy(table_hbm.at[idx_v], rows_v, sem).wait()
            pltpu.sync_copy(rows_v, out_hbm.at[pl.ds(off, CHUNK)])
            return carry

        lax.fori_loop(0, N_CHUNK, body, 0)

    return gather_k(emb, idx_flat)


def _sigmoid(z):
    return 0.5 * jnp.tanh(0.5 * z) + 0.5


def _cell(xh, c_old, w_r, b_r):
    gates = lax.dot_general(xh, w_r[...], (((1,), (1,)), ((), ())),
                            preferred_element_type=jnp.float32) + b_r[...]
    i = _sigmoid(gates[:, 0:H])
    f = _sigmoid(gates[:, H:2 * H])
    g = jnp.tanh(gates[:, 2 * H:3 * H])
    o = _sigmoid(gates[:, 3 * H:4 * H])
    c = f * c_old + i * g
    h = o * jnp.tanh(c)
    return h, c


def _fused_lstm(e, w0, b0, w1, b1, w2, b2, ln_g, ln_b, interpret=False):
    bf = jnp.bfloat16

    def body(x_ref, w0_r, b0_r, w1_r, b1_r, w2_r, b2_r, lng_r, lnb_r,
             hT, cT, nrm16,
             h0s, h1s, h2s, c0s, c1s, c2s):
        s = pl.program_id(0)

        @pl.when(s == 0)
        def _init():
            for r in (h0s, h1s, h2s):
                r[...] = jnp.zeros((B, H), bf)
            for r in (c0s, c1s, c2s):
                r[...] = jnp.zeros((B, H), jnp.float32)

        h0o = h0s[...]
        h1o = h1s[...]

        ha, ca = _cell(jnp.concatenate([x_ref[0].astype(bf), h0o], axis=1),
                       c0s[...], w0_r, b0_r)
        hb, cb = _cell(jnp.concatenate([h0o, h1o], axis=1),
                       c1s[...], w1_r, b1_r)
        hc, cc = _cell(jnp.concatenate([h1o, h2s[...]], axis=1),
                       c2s[...], w2_r, b2_r)

        h0s[...] = ha.astype(bf)
        c0s[...] = ca
        h1s[...] = hb.astype(bf)
        c1s[...] = cb
        h2s[...] = hc.astype(bf)
        c2s[...] = cc

        @pl.when(s == 0)
        def _rz0():
            for r in (h1s, h2s):
                r[...] = jnp.zeros((B, H), bf)
            for r in (c1s, c2s):
                r[...] = jnp.zeros((B, H), jnp.float32)

        @pl.when(s == 1)
        def _rz1():
            h2s[...] = jnp.zeros((B, H), bf)
            c2s[...] = jnp.zeros((B, H), jnp.float32)

        @pl.when(s == T - 1)
        def _cap0():
            hT[0] = ha
            cT[0] = ca

        @pl.when(s == T)
        def _cap1():
            hT[1] = hb
            cT[1] = cb

        @pl.when(s == T + 1)
        def _fin():
            hT[2] = hc
            cT[2] = cc
            mu = jnp.mean(hc, axis=-1, keepdims=True)
            var = jnp.mean((hc - mu) ** 2, axis=-1, keepdims=True)
            n = ((hc - mu) * lax.rsqrt(var + 1e-5)
                 * lng_r[...] + lnb_r[...])
            nrm16[...] = n.astype(bf)

    const = lambda s: (0, 0)
    return pl.pallas_call(
        body,
        grid=(T + 2,),
        in_specs=[
            pl.BlockSpec((1, B, E), lambda s: (jnp.minimum(s, T - 1), 0, 0)),
            pl.BlockSpec((G4, E + H), const),
            pl.BlockSpec((1, G4), const),
            pl.BlockSpec((G4, 2 * H), const),
            pl.BlockSpec((1, G4), const),
            pl.BlockSpec((G4, 2 * H), const),
            pl.BlockSpec((1, G4), const),
            pl.BlockSpec((1, H), const),
            pl.BlockSpec((1, H), const),
        ],
        out_specs=[
            pl.BlockSpec((L, B, H), lambda s: (0, 0, 0)),
            pl.BlockSpec((L, B, H), lambda s: (0, 0, 0)),
            pl.BlockSpec((B, H), const),
        ],
        out_shape=[
            jax.ShapeDtypeStruct((L, B, H), jnp.float32),
            jax.ShapeDtypeStruct((L, B, H), jnp.float32),
            jax.ShapeDtypeStruct((B, H), jnp.bfloat16),
        ],
        scratch_shapes=[pltpu.VMEM((B, H), bf)] * 3
        + [pltpu.VMEM((B, H), jnp.float32)] * 3,
        interpret=interpret,
    )(e, w0, b0, w1, b1, w2, b2, ln_g, ln_b)


VB = 4096
NVB = -(-V // VB)
VPAD = NVB * VB


def _head(normed, lin_W, lin_b_pad, interpret=False):
    def body(n_ref, w_ref, b_ref, out_ref):
        acc = lax.dot_general(w_ref[...].astype(jnp.bfloat16),
                              n_ref[...],
                              (((1,), (1,)), ((), ())),
                              preferred_element_type=jnp.float32)
        out_ref[...] = acc + jnp.transpose(b_ref[0])

    return pl.pallas_call(
        body,
        grid=(NVB,),
        in_specs=[
            pl.BlockSpec((B, H), lambda i: (0, 0)),
            pl.BlockSpec((VB, H), lambda i: (i, 0)),
            pl.BlockSpec((1, 1, VB), lambda i: (i, 0, 0)),
        ],
        out_specs=pl.BlockSpec((VB, B), lambda i: (i, 0)),
        out_shape=jax.ShapeDtypeStruct((V, B), jnp.float32),
        interpret=interpret,
    )(normed, lin_W, lin_b_pad)


def kernel(x, emb, W_ih0, W_hh0, b_ih0, b_hh0, W_ih1, W_hh1, b_ih1, b_hh1,
           W_ih2, W_hh2, b_ih2, b_hh2, ln_g, ln_b, lin_W, lin_b):
    idx_flat = x.T.reshape(BT)
    e = _sc_gather(emb, idx_flat).reshape(T, B, E)

    bf = jnp.bfloat16
    w0 = jnp.concatenate([W_ih0, W_hh0], axis=1).astype(bf)
    w1 = jnp.concatenate([W_ih1, W_hh1], axis=1).astype(bf)
    w2 = jnp.concatenate([W_ih2, W_hh2], axis=1).astype(bf)
    hT, cT, normed16 = _fused_lstm(
        e,
        w0, (b_ih0 + b_hh0).reshape(1, G4),
        w1, (b_ih1 + b_hh1).reshape(1, G4),
        w2, (b_ih2 + b_hh2).reshape(1, G4),
        ln_g.reshape(1, H), ln_b.reshape(1, H))

    lin_b_pad = jnp.zeros((VPAD,), jnp.float32).at[:V].set(lin_b)
    logits = _head(normed16, lin_W, lin_b_pad.reshape(NVB, 1, VB)).T

    return logits, (hT, cT)


# speedup vs baseline: 1.0479x; 1.0094x over previous
import functools

import jax
import jax.numpy as jnp
from jax import lax
from jax.experimental import pallas as pl
from jax.experimental.pallas import tpu as pltpu
from jax.experimental.pallas import tpu_sc as plsc

V = 100000
E = 128
H = 256
B = 1024
T = 50
L = 3
G4 = 4 * H

NC, NS = 2, 16
NW = NC * NS
BT = B * T
B_PER_W = BT // NW
CHUNK = 800
N_CHUNK = B_PER_W // CHUNK


def _sc_gather(emb, idx_flat):
    mesh = plsc.VectorSubcoreMesh(core_axis_name="c", subcore_axis_name="s")

    @functools.partial(
        pl.kernel,
        mesh=mesh,
        out_type=jax.ShapeDtypeStruct((BT, E), jnp.float32),
        scratch_types=[
            pltpu.VMEM((CHUNK,), jnp.int32),
            pltpu.VMEM((CHUNK, E), jnp.float32),
            pltpu.SemaphoreType.DMA,
        ],
    )
    def gather_k(table_hbm, idx_hbm, out_hbm, idx_v, rows_v, sem):
        wid = lax.axis_index("s") * NC + lax.axis_index("c")
        base = wid * B_PER_W

        def body(j, carry):
            off = base + j * CHUNK
            pltpu.sync_copy(idx_hbm.at[pl.ds(off, CHUNK)], idx_v)
            pltpu.async_copy(table_hbm.at[idx_v], rows_v, sem).wait()
            pltpu.sync_copy(rows_v, out_hbm.at[pl.ds(off, CHUNK)])
            return carry

        lax.fori_loop(0, N_CHUNK, body, 0)

    return gather_k(emb, idx_flat)


def _sigmoid(z):
    return 0.5 * jnp.tanh(0.5 * z) + 0.5


def _cell(xh, c_old, w_r, b_r):
    gates = lax.dot_general(xh, w_r[...], (((1,), (1,)), ((), ())),
                            preferred_element_type=jnp.float32) + b_r[...]
    i = _sigmoid(gates[:, 0:H])
    f = _sigmoid(gates[:, H:2 * H])
    g = jnp.tanh(gates[:, 2 * H:3 * H])
    o = _sigmoid(gates[:, 3 * H:4 * H])
    c = f * c_old + i * g
    h = o * jnp.tanh(c)
    return h, c


def _fused_lstm(e, w0, b0, w1, b1, w2, b2, ln_g, ln_b, interpret=False):
    bf = jnp.bfloat16

    def body(x_ref, w0_r, b0_r, w1_r, b1_r, w2_r, b2_r, lng_r, lnb_r,
             hT, cT, nrm16,
             h0s, h1s, h2s, c0s, c1s, c2s):
        s = pl.program_id(0)

        @pl.when(s == 0)
        def _init():
            for r in (h0s, h1s, h2s):
                r[...] = jnp.zeros((B, H), bf)
            for r in (c0s, c1s, c2s):
                r[...] = jnp.zeros((B, H), jnp.float32)

        h0o = h0s[...]
        h1o = h1s[...]

        ha, ca = _cell(jnp.concatenate([x_ref[0].astype(bf), h0o], axis=1),
                       c0s[...], w0_r, b0_r)
        hb, cb = _cell(jnp.concatenate([h0o, h1o], axis=1),
                       c1s[...], w1_r, b1_r)
        hc, cc = _cell(jnp.concatenate([h1o, h2s[...]], axis=1),
                       c2s[...], w2_r, b2_r)

        h0s[...] = ha.astype(bf)
        c0s[...] = ca
        h1s[...] = hb.astype(bf)
        c1s[...] = cb
        h2s[...] = hc.astype(bf)
        c2s[...] = cc

        @pl.when(s == 0)
        def _rz0():
            for r in (h1s, h2s):
                r[...] = jnp.zeros((B, H), bf)
            for r in (c1s, c2s):
                r[...] = jnp.zeros((B, H), jnp.float32)

        @pl.when(s == 1)
        def _rz1():
            h2s[...] = jnp.zeros((B, H), bf)
            c2s[...] = jnp.zeros((B, H), jnp.float32)

        @pl.when(s == T - 1)
        def _cap0():
            hT[0] = ha
            cT[0] = ca

        @pl.when(s == T)
        def _cap1():
            hT[1] = hb
            cT[1] = cb

        @pl.when(s == T + 1)
        def _fin():
            hT[2] = hc
            cT[2] = cc
            mu = jnp.mean(hc, axis=-1, keepdims=True)
            var = jnp.mean((hc - mu) ** 2, axis=-1, keepdims=True)
            n = ((hc - mu) * lax.rsqrt(var + 1e-5)
                 * lng_r[...] + lnb_r[...])
            nrm16[...] = n.astype(bf)

    const = lambda s: (0, 0)
    return pl.pallas_call(
        body,
        grid=(T + 2,),
        in_specs=[
            pl.BlockSpec((1, B, E), lambda s: (jnp.minimum(s, T - 1), 0, 0)),
            pl.BlockSpec((G4, E + H), const),
            pl.BlockSpec((1, G4), const),
            pl.BlockSpec((G4, 2 * H), const),
            pl.BlockSpec((1, G4), const),
            pl.BlockSpec((G4, 2 * H), const),
            pl.BlockSpec((1, G4), const),
            pl.BlockSpec((1, H), const),
            pl.BlockSpec((1, H), const),
        ],
        out_specs=[
            pl.BlockSpec((L, B, H), lambda s: (0, 0, 0)),
            pl.BlockSpec((L, B, H), lambda s: (0, 0, 0)),
            pl.BlockSpec((B, H), const),
        ],
        out_shape=[
            jax.ShapeDtypeStruct((L, B, H), jnp.float32),
            jax.ShapeDtypeStruct((L, B, H), jnp.float32),
            jax.ShapeDtypeStruct((B, H), jnp.bfloat16),
        ],
        scratch_shapes=[pltpu.VMEM((B, H), bf)] * 3
        + [pltpu.VMEM((B, H), jnp.float32)] * 3,
        interpret=interpret,
    )(e, w0, b0, w1, b1, w2, b2, ln_g, ln_b)


VB = 4096
NVB = -(-V // VB)
VPAD = NVB * VB


def _head(normed, lin_W, lin_b_pad, interpret=False):
    def body(n_ref, w_ref, b_ref, out_ref):
        acc = lax.dot_general(w_ref[...].astype(jnp.bfloat16),
                              n_ref[...],
                              (((1,), (1,)), ((), ())),
                              preferred_element_type=jnp.float32)
        out_ref[...] = acc + jnp.transpose(b_ref[0])

    return pl.pallas_call(
        body,
        grid=(NVB,),
        in_specs=[
            pl.BlockSpec((B, H), lambda i: (0, 0)),
            pl.BlockSpec((VB, H), lambda i: (i, 0)),
            pl.BlockSpec((1, 1, VB), lambda i: (i, 0, 0)),
        ],
        out_specs=pl.BlockSpec((VB, B), lambda i: (i, 0)),
        out_shape=jax.ShapeDtypeStruct((V, B), jnp.float32),
        interpret=interpret,
    )(normed, lin_W, lin_b_pad)


def kernel(x, emb, W_ih0, W_hh0, b_ih0, b_hh0, W_ih1, W_hh1, b_ih1, b_hh1,
           W_ih2, W_hh2, b_ih2, b_hh2, ln_g, ln_b, lin_W, lin_b):
    idx_flat = x.T.reshape(BT)
    e = _sc_gather(emb, idx_flat).reshape(T, B, E)

    bf = jnp.bfloat16
    w0 = jnp.concatenate([W_ih0, W_hh0], axis=1).astype(bf)
    w1 = jnp.concatenate([W_ih1, W_hh1], axis=1).astype(bf)
    w2 = jnp.concatenate([W_ih2, W_hh2], axis=1).astype(bf)
    hT, cT, normed16 = _fused_lstm(
        e,
        w0, (b_ih0 + b_hh0).reshape(1, G4),
        w1, (b_ih1 + b_hh1).reshape(1, G4),
        w2, (b_ih2 + b_hh2).reshape(1, G4),
        ln_g.reshape(1, H), ln_b.reshape(1, H))

    lin_b_pad = jnp.zeros((VPAD,), jnp.float32).at[:V].set(lin_b)
    logits = _head(normed16, lin_W, lin_b_pad.reshape(NVB, 1, VB)).T

    return logits, (hT, cT)
